# Initial kernel scaffold; baseline (speedup 1.0000x reference)
import jax
import jax.numpy as jnp
from jax import lax
from jax.experimental import pallas as pl
from jax.experimental.pallas import tpu as pltpu
from jax.experimental.pallas import tpu_sc as plsc

N_NODE = 10000
N_HE = 10000
D = 128
R = 10016
NC = 2
NS = 16
NW = NC * NS
CHUNK = 128
NNZ = 320000
K = 80
NNZ_PAD = NW * K * CHUNK
KD = NNZ_PAD // (NS * CHUNK)
RPS = R // NS

_F32 = jnp.float32
_MESH = plsc.VectorSubcoreMesh(core_axis_name="c", subcore_axis_name="s")


def _zero_rows(ref, nrows, width):
    @pl.loop(0, nrows)
    def _(i):
        @pl.loop(0, width, step=16)
        def _(k):
            ref.at[i, pl.ds(k, 16)][...] = jnp.zeros((16,), _F32)


def _deg_body(idx_hbm, out_hbm, idxv, onesv, zbuf, accum):
    c = lax.axis_index("c")
    s = lax.axis_index("s")

    @pl.loop(0, CHUNK)
    def _(i):
        onesv.at[i, pl.ds(0, 16)][...] = jnp.ones((16,), _F32)

    _zero_rows(zbuf, RPS, 16)
    pltpu.sync_copy(zbuf, accum.at[pl.ds(s * RPS, RPS)])
    pltpu.sync_copy(idx_hbm.at[c, s], idxv)
    plsc.subcore_barrier()

    @pl.loop(0, KD)
    def _(j):
        pltpu.sync_copy(onesv, accum.at[idxv.at[j]], add=True)

    plsc.subcore_barrier()
    pltpu.sync_copy(accum.at[pl.ds(s * RPS, RPS)],
                    out_hbm.at[c, pl.ds(s * RPS, RPS)])


_deg = pl.kernel(
    _deg_body,
    out_type=jax.ShapeDtypeStruct((NC, R, 16), _F32),
    mesh=_MESH,
    scratch_types=[
        pltpu.VMEM((KD, CHUNK), jnp.int32),
        pltpu.VMEM((CHUNK, 16), _F32),
        pltpu.VMEM((RPS, 16), _F32),
        pltpu.VMEM_SHARED((R, 16), _F32),
    ],
)


def _pass_body(table_hbm, src_hbm, dst_hbm, out_hbm, srcv, dstv, rows, zbuf,
               accum, sem):
    c = lax.axis_index("c")
    s = lax.axis_index("s")
    wid = s * NC + c

    _zero_rows(zbuf, CHUNK, D)
    for off, n in ((0, 128), (128, 128), (256, 128), (384, 128), (512, 114)):
        pltpu.sync_copy(zbuf.at[pl.ds(0, n)],
                        accum.at[pl.ds(s * RPS + off, n)])
    pltpu.sync_copy(src_hbm.at[wid], srcv)
    pltpu.sync_copy(dst_hbm.at[wid], dstv)
    plsc.subcore_barrier()

    @pl.loop(0, K)
    def _(j):
        pltpu.async_copy(table_hbm.at[srcv.at[j]], rows, sem).wait()
        pltpu.sync_copy(rows, accum.at[dstv.at[j]], add=True)

    plsc.subcore_barrier()
    for off, n in ((0, 128), (128, 128), (256, 128), (384, 128), (512, 114)):
        pltpu.sync_copy(accum.at[pl.ds(s * RPS + off, n)],
                        out_hbm.at[c, pl.ds(s * RPS + off, n)])


_pass = pl.kernel(
    _pass_body,
    out_type=jax.ShapeDtypeStruct((NC, R, D), _F32),
    mesh=_MESH,
    scratch_types=[
        pltpu.VMEM((K, CHUNK), jnp.int32),
        pltpu.VMEM((K, CHUNK), jnp.int32),
        pltpu.VMEM((CHUNK, D), _F32),
        pltpu.VMEM((CHUNK, D), _F32),
        pltpu.VMEM_SHARED((R, D), _F32),
        pltpu.SemaphoreType.DMA,
    ],
)


def _mm_body(x_ref, w_ref, o_ref):
    o_ref[...] = jnp.dot(x_ref[...], w_ref[...],
                         preferred_element_type=_F32)


_mm = pl.pallas_call(_mm_body, out_shape=jax.ShapeDtypeStruct((R, D), _F32))


def _scale_he_body(p_ref, hist_ref, o_ref):
    b = hist_ref[:, 0:1]
    binv = jnp.where(b > 0, 1.0 / b, 0.0)
    o_ref[...] = binv * (p_ref[0] + p_ref[1])


_scale_he = pl.pallas_call(
    _scale_he_body, out_shape=jax.ShapeDtypeStruct((R, D), _F32))


def _fuse_body(q_ref, hist_ref, b_ref, w_ref, o_ref):
    d = hist_ref[:, 0:1]
    dinv = jnp.where(d > 0, 1.0 / d, 0.0)
    h = dinv * (q_ref[0] + q_ref[1]) + b_ref[...]
    h = jnp.where(h > 0, h, jnp.expm1(h))
    o_ref[...] = jnp.dot(h, w_ref[...], preferred_element_type=_F32)


_fuse = pl.pallas_call(
    _fuse_body, out_shape=jax.ShapeDtypeStruct((R, D), _F32))


def _final_body(q_ref, hist_ref, b_ref, o_ref):
    d = hist_ref[:, 0:1]
    dinv = jnp.where(d > 0, 1.0 / d, 0.0)
    h = dinv * (q_ref[0] + q_ref[1]) + b_ref[...]
    o_ref[...] = jnp.where(h > 0, h, jnp.expm1(h))


_final = pl.pallas_call(
    _final_body, out_shape=jax.ShapeDtypeStruct((R, D), _F32))


def kernel(x, edge_index, W1, b1, W2, b2):
    node = edge_index[0].astype(jnp.int32)
    he = edge_index[1].astype(jnp.int32)
    npad = NNZ_PAD - NNZ
    node_p = jnp.concatenate([node, jnp.full((npad,), N_NODE, jnp.int32)])
    he_p = jnp.concatenate([he, jnp.full((npad,), N_HE, jnp.int32)])
    node_w = node_p.reshape(NW, K, CHUNK)
    he_w = he_p.reshape(NW, K, CHUNK)
    deg_idx = jnp.stack([node_p.reshape(NS, KD, CHUNK),
                         he_p.reshape(NS, KD, CHUNK)])
    xp = jnp.pad(x, ((0, R - N_NODE), (0, 0)))
    b1r = b1.reshape(1, D)
    b2r = b2.reshape(1, D)

    hist = _deg(deg_idx)
    hist_node = hist[0]
    hist_he = hist[1]

    xw1 = _mm(xp, W1)
    p1 = _pass(xw1, node_w, he_w)
    he1 = _scale_he(p1, hist_he)
    q1 = _pass(he1, he_w, node_w)
    xw2 = _fuse(q1, hist_node, b1r, W2)
    p2 = _pass(xw2, node_w, he_w)
    he2 = _scale_he(p2, hist_he)
    q2 = _pass(he2, he_w, node_w)
    out = _final(q2, hist_node, b2r)
    return out[:N_NODE]


# speedup vs baseline: 4.3765x; 4.3765x over previous
import jax
import jax.numpy as jnp
from jax import lax
from jax.experimental import pallas as pl
from jax.experimental.pallas import tpu as pltpu
from jax.experimental.pallas import tpu_sc as plsc

N_NODE = 10000
N_HE = 10000
D = 128
R = 10112
NC = 2
NS = 16
NW = NC * NS
CHUNK = 128
NNZ = 320000
K = 80
NNZ_PAD = NW * K * CHUNK
KD = NNZ_PAD // (NS * CHUNK)
RPS = R // NS
_SLICES = ((0, 128), (128, 128), (256, 128), (384, 128), (512, 120))

_F32 = jnp.float32
_I32 = jnp.int32
_MESH = plsc.VectorSubcoreMesh(core_axis_name="c", subcore_axis_name="s")


def _zero_rows(ref, nrows, width):
    @pl.loop(0, nrows)
    def _(i):
        @pl.loop(0, width, step=16)
        def _(k):
            ref.at[i, pl.ds(k, 16)][...] = jnp.zeros((16,), _F32)


def _histk_body(idx_hbm, out_hbm, idxv, buf, accum):
    c = lax.axis_index("c")
    s = lax.axis_index("s")
    wid = s * NC + c

    _zero_rows(buf, CHUNK, D)
    for off, n in _SLICES:
        pltpu.sync_copy(buf.at[pl.ds(0, n)],
                        accum.at[pl.ds(s * RPS + off, n)])
    pltpu.sync_copy(idx_hbm.at[wid], idxv)

    @pl.loop(0, CHUNK)
    def _(i):
        @pl.loop(0, D, step=16)
        def _(kk):
            buf.at[i, pl.ds(kk, 16)][...] = jnp.ones((16,), _F32)

    plsc.subcore_barrier()

    @pl.loop(0, K)
    def _(j):
        pltpu.sync_copy(buf, accum.at[idxv.at[j]], add=True)

    plsc.subcore_barrier()
    for off, n in _SLICES:
        pltpu.sync_copy(accum.at[pl.ds(s * RPS + off, n)],
                        out_hbm.at[c, pl.ds(s * RPS + off, n)])


_histk = pl.kernel(
    _histk_body,
    out_type=jax.ShapeDtypeStruct((NC, R, D), _F32),
    mesh=_MESH,
    scratch_types=[
        pltpu.VMEM((K, CHUNK), _I32),
        pltpu.VMEM((CHUNK, D), _F32),
        pltpu.VMEM_SHARED((R, D), _F32),
    ],
)


def _pass_body(table_hbm, src_hbm, dst_hbm, out_hbm, srcv, dstv, rows,
               accum, sem):
    c = lax.axis_index("c")
    s = lax.axis_index("s")
    wid = s * NC + c

    _zero_rows(rows, CHUNK, D)
    for off, n in _SLICES:
        pltpu.sync_copy(rows.at[pl.ds(0, n)],
                        accum.at[pl.ds(s * RPS + off, n)])
    pltpu.sync_copy(src_hbm.at[wid], srcv)
    pltpu.sync_copy(dst_hbm.at[wid], dstv)
    plsc.subcore_barrier()

    @pl.loop(0, K)
    def _(j):
        pltpu.async_copy(table_hbm.at[srcv.at[j]], rows, sem).wait()
        pltpu.sync_copy(rows, accum.at[dstv.at[j]], add=True)

    plsc.subcore_barrier()
    for off, n in _SLICES:
        pltpu.sync_copy(accum.at[pl.ds(s * RPS + off, n)],
                        out_hbm.at[c, pl.ds(s * RPS + off, n)])


_pass = pl.kernel(
    _pass_body,
    out_type=jax.ShapeDtypeStruct((NC, R, D), _F32),
    mesh=_MESH,
    scratch_types=[
        pltpu.VMEM((K, CHUNK), _I32),
        pltpu.VMEM((K, CHUNK), _I32),
        pltpu.VMEM((CHUNK, D), _F32),
        pltpu.VMEM_SHARED((R, D), _F32),
        pltpu.SemaphoreType.DMA,
    ],
)


def _mm_body(x_ref, w_ref, o_ref):
    o_ref[...] = jnp.dot(x_ref[...], w_ref[...],
                         preferred_element_type=_F32)


_mm = pl.pallas_call(_mm_body, out_shape=jax.ShapeDtypeStruct((R, D), _F32))


def _scale_he_body(p_ref, hist_ref, o_ref):
    b = hist_ref[0, :, 0:1] + hist_ref[1, :, 0:1]
    binv = jnp.where(b > 0, 1.0 / b, 0.0)
    o_ref[...] = binv * (p_ref[0] + p_ref[1])


_scale_he = pl.pallas_call(
    _scale_he_body, out_shape=jax.ShapeDtypeStruct((R, D), _F32))


def _fuse_body(q_ref, hist_ref, b_ref, w_ref, o_ref):
    d = hist_ref[0, :, 0:1] + hist_ref[1, :, 0:1]
    dinv = jnp.where(d > 0, 1.0 / d, 0.0)
    h = dinv * (q_ref[0] + q_ref[1]) + b_ref[...]
    h = jnp.where(h > 0, h, jnp.exp(h) - 1.0)
    o_ref[...] = jnp.dot(h, w_ref[...], preferred_element_type=_F32)


_fuse = pl.pallas_call(
    _fuse_body, out_shape=jax.ShapeDtypeStruct((R, D), _F32))


def _final_body(q_ref, hist_ref, b_ref, o_ref):
    d = hist_ref[0, :, 0:1] + hist_ref[1, :, 0:1]
    dinv = jnp.where(d > 0, 1.0 / d, 0.0)
    h = dinv * (q_ref[0] + q_ref[1]) + b_ref[...]
    o_ref[...] = jnp.where(h > 0, h, jnp.exp(h) - 1.0)


_final = pl.pallas_call(
    _final_body, out_shape=jax.ShapeDtypeStruct((R, D), _F32))


def kernel(x, edge_index, W1, b1, W2, b2):
    node = edge_index[0].astype(_I32)
    he = edge_index[1].astype(_I32)
    npad = NNZ_PAD - NNZ
    node_p = jnp.concatenate([node, jnp.full((npad,), N_NODE, _I32)])
    he_p = jnp.concatenate([he, jnp.full((npad,), N_HE, _I32)])
    node_w = node_p.reshape(NW, K, CHUNK)
    he_w = he_p.reshape(NW, K, CHUNK)
    xp = jnp.pad(x, ((0, R - N_NODE), (0, 0)))
    b1r = b1.reshape(1, D)
    b2r = b2.reshape(1, D)

    hist_node = _histk(node_w)
    hist_he = _histk(he_w)

    xw1 = _mm(xp, W1)
    p1 = _pass(xw1, node_w, he_w)
    he1 = _scale_he(p1, hist_he)
    q1 = _pass(he1, he_w, node_w)
    xw2 = _fuse(q1, hist_node, b1r, W2)
    p2 = _pass(xw2, node_w, he_w)
    he2 = _scale_he(p2, hist_he)
    q2 = _pass(he2, he_w, node_w)
    out = _final(q2, hist_node, b2r)
    return out[:N_NODE]


# speedup vs baseline: 5.8941x; 1.3467x over previous
import jax
import jax.numpy as jnp
from jax import lax
from jax.experimental import pallas as pl
from jax.experimental.pallas import tpu as pltpu
from jax.experimental.pallas import tpu_sc as plsc

N_NODE = 10000
N_HE = 10000
D = 128
R = 10112
NC = 2
NS = 16
NW = NC * NS
CHUNK = 128
NNZ = 320000
K = 80
NNZ_PAD = NW * K * CHUNK
RPS = R // NS
_SLICES = ((0, 128), (128, 128), (256, 128), (384, 128), (512, 120))

_F32 = jnp.float32
_I32 = jnp.int32
_MESH = plsc.VectorSubcoreMesh(core_axis_name="c", subcore_axis_name="s")


def _zero_rows(ref, nrows, width):
    @pl.loop(0, nrows)
    def _(i):
        @pl.loop(0, width, step=16)
        def _(k):
            ref.at[i, pl.ds(k, 16)][...] = jnp.zeros((16,), _F32)


def _histk_body(idx_hbm, dep_hbm, out_hbm, idxv, buf, accum):
    c = lax.axis_index("c")
    s = lax.axis_index("s")
    wid = s * NC + c

    _zero_rows(buf, CHUNK, D)
    for off, n in _SLICES:
        pltpu.sync_copy(buf.at[pl.ds(0, n)],
                        accum.at[pl.ds(s * RPS + off, n)])
    pltpu.sync_copy(idx_hbm.at[wid], idxv)

    @pl.loop(0, CHUNK)
    def _(i):
        @pl.loop(0, D, step=16)
        def _(kk):
            buf.at[i, pl.ds(kk, 16)][...] = jnp.ones((16,), _F32)

    plsc.subcore_barrier()

    @pl.loop(0, K)
    def _(j):
        pltpu.sync_copy(buf, accum.at[idxv.at[j]], add=True)

    plsc.subcore_barrier()
    for off, n in _SLICES:
        pltpu.sync_copy(accum.at[pl.ds(s * RPS + off, n)],
                        out_hbm.at[c, pl.ds(s * RPS + off, n)])


_histk = pl.kernel(
    _histk_body,
    out_type=jax.ShapeDtypeStruct((NC, R, D), _F32),
    mesh=_MESH,
    scratch_types=[
        pltpu.VMEM((K, CHUNK), _I32),
        pltpu.VMEM((CHUNK, D), _F32),
        pltpu.VMEM_SHARED((R, D), _F32),
    ],
)


def _pass_body(table_hbm, src_hbm, dst_hbm, dep_hbm, out_hbm, dstv, istage,
               rows, accum, isem0, isem1, gsem0, gsem1):
    c = lax.axis_index("c")
    s = lax.axis_index("s")
    wid = s * NC + c

    @pl.loop(0, CHUNK)
    def _(i):
        @pl.loop(0, D, step=16)
        def _(kk):
            rows.at[0, i, pl.ds(kk, 16)][...] = jnp.zeros((16,), _F32)

    for off, n in _SLICES:
        pltpu.sync_copy(rows.at[0, pl.ds(0, n)],
                        accum.at[pl.ds(s * RPS + off, n)])
    pltpu.sync_copy(dst_hbm.at[wid], dstv)
    plsc.subcore_barrier()

    pltpu.sync_copy(src_hbm.at[wid, 0], istage.at[0])
    pltpu.async_copy(table_hbm.at[istage.at[0]], rows.at[0], gsem0)
    pltpu.async_copy(src_hbm.at[wid, 1], istage.at[1], isem1)

    @pl.loop(0, K // 2)
    def _(jj):
        j0 = 2 * jj
        pltpu.make_async_copy(src_hbm.at[wid, j0 + 1], istage.at[1],
                              isem1).wait()
        pltpu.async_copy(table_hbm.at[istage.at[1]], rows.at[1], gsem1)
        pltpu.make_async_copy(table_hbm.at[istage.at[0]], rows.at[0],
                              gsem0).wait()

        @pl.when(j0 + 2 < K)
        def _():
            pltpu.async_copy(src_hbm.at[wid, j0 + 2], istage.at[0], isem0)

        pltpu.sync_copy(rows.at[0], accum.at[dstv.at[j0]], add=True)

        @pl.when(j0 + 2 < K)
        def _():
            pltpu.make_async_copy(src_hbm.at[wid, j0 + 2], istage.at[0],
                                  isem0).wait()
            pltpu.async_copy(table_hbm.at[istage.at[0]], rows.at[0], gsem0)

        pltpu.make_async_copy(table_hbm.at[istage.at[1]], rows.at[1],
                              gsem1).wait()

        @pl.when(j0 + 3 < K)
        def _():
            pltpu.async_copy(src_hbm.at[wid, j0 + 3], istage.at[1], isem1)

        pltpu.sync_copy(rows.at[1], accum.at[dstv.at[j0 + 1]], add=True)

    plsc.subcore_barrier()
    for off, n in _SLICES:
        pltpu.sync_copy(accum.at[pl.ds(s * RPS + off, n)],
                        out_hbm.at[c, pl.ds(s * RPS + off, n)])


_pass = pl.kernel(
    _pass_body,
    out_type=jax.ShapeDtypeStruct((NC, R, D), _F32),
    mesh=_MESH,
    scratch_types=[
        pltpu.VMEM((K, CHUNK), _I32),
        pltpu.VMEM((2, CHUNK), _I32),
        pltpu.VMEM((2, CHUNK, D), _F32),
        pltpu.VMEM_SHARED((R, D), _F32),
        pltpu.SemaphoreType.DMA,
        pltpu.SemaphoreType.DMA,
        pltpu.SemaphoreType.DMA,
        pltpu.SemaphoreType.DMA,
    ],
)


def _mm_body(x_ref, w_ref, o_ref):
    o_ref[...] = jnp.dot(x_ref[...], w_ref[...],
                         preferred_element_type=_F32)


_mm = pl.pallas_call(_mm_body, out_shape=jax.ShapeDtypeStruct((R, D), _F32))


def _scale_he_body(p_ref, hist_ref, o_ref):
    b = hist_ref[0, :, 0:1] + hist_ref[1, :, 0:1]
    binv = jnp.where(b > 0, 1.0 / b, 0.0)
    o_ref[...] = binv * (p_ref[0] + p_ref[1])


_scale_he = pl.pallas_call(
    _scale_he_body, out_shape=jax.ShapeDtypeStruct((R, D), _F32))


def _fuse_body(q_ref, hist_ref, b_ref, w_ref, o_ref):
    d = hist_ref[0, :, 0:1] + hist_ref[1, :, 0:1]
    dinv = jnp.where(d > 0, 1.0 / d, 0.0)
    h = dinv * (q_ref[0] + q_ref[1]) + b_ref[...]
    h = jnp.where(h > 0, h, jnp.exp(h) - 1.0)
    o_ref[...] = jnp.dot(h, w_ref[...], preferred_element_type=_F32)


_fuse = pl.pallas_call(
    _fuse_body, out_shape=jax.ShapeDtypeStruct((R, D), _F32))


def _final_body(q_ref, hist_ref, b_ref, o_ref):
    d = hist_ref[0, :, 0:1] + hist_ref[1, :, 0:1]
    dinv = jnp.where(d > 0, 1.0 / d, 0.0)
    h = dinv * (q_ref[0] + q_ref[1]) + b_ref[...]
    o_ref[...] = jnp.where(h > 0, h, jnp.exp(h) - 1.0)


_final = pl.pallas_call(
    _final_body, out_shape=jax.ShapeDtypeStruct((R, D), _F32))


def kernel(x, edge_index, W1, b1, W2, b2):
    node = edge_index[0].astype(_I32)
    he = edge_index[1].astype(_I32)
    npad = NNZ_PAD - NNZ
    node_p = jnp.concatenate([node, jnp.full((npad,), N_NODE, _I32)])
    he_p = jnp.concatenate([he, jnp.full((npad,), N_HE, _I32)])
    node_w = node_p.reshape(NW, K, CHUNK)
    he_w = he_p.reshape(NW, K, CHUNK)
    xp = jnp.pad(x, ((0, R - N_NODE), (0, 0)))
    b1r = b1.reshape(1, D)
    b2r = b2.reshape(1, D)

    hist_node = _histk(node_w, W1)
    hist_he = _histk(he_w, hist_node)

    xw1 = _mm(xp, W1)
    p1 = _pass(xw1, node_w, he_w, hist_he)
    he1 = _scale_he(p1, hist_he)
    q1 = _pass(he1, he_w, node_w, p1)
    xw2 = _fuse(q1, hist_node, b1r, W2)
    p2 = _pass(xw2, node_w, he_w, q1)
    he2 = _scale_he(p2, hist_he)
    q2 = _pass(he2, he_w, node_w, p2)
    out = _final(q2, hist_node, b2r)
    return out[:N_NODE]
